# Initial kernel scaffold; baseline (speedup 1.0000x reference)
import jax
import jax.numpy as jnp
from jax.experimental import pallas as pl


def kernel(X, Q, ranks):
    raise NotImplementedError("write your pallas kernel here")


# speedup vs baseline: 2.6726x; 2.6726x over previous
import functools

import jax
import jax.numpy as jnp
from jax import lax
from jax.experimental import pallas as pl
from jax.experimental.pallas import tpu as pltpu
from jax.experimental.pallas import tpu_sc as plsc

_M = 30
_K = 20
_BETA = 0.15
_MP = 32
_BQ = 8
_NW = 32
_NEG = -1e30


def _sc_gather(X, idx_sc):
    mesh = plsc.VectorSubcoreMesh(core_axis_name="c", subcore_axis_name="s")

    @functools.partial(
        pl.kernel,
        mesh=mesh,
        compiler_params=pltpu.CompilerParams(use_tc_tiling_on_sc=False),
        out_type=jax.ShapeDtypeStruct((_NW * 1024, 64), jnp.float32),
        scratch_types=[
            pltpu.VMEM((8, 128), jnp.int32),
            pltpu.VMEM((1024, 64), jnp.float32),
            pltpu.SemaphoreType.DMA,
        ],
    )
    def gather_k(x_hbm, idx_hbm, out_hbm, idx_v, rows_v, sem):
        wid = lax.axis_index("s") * 2 + lax.axis_index("c")
        pltpu.sync_copy(idx_hbm.at[wid], idx_v)
        descs = [
            pltpu.async_copy(
                x_hbm.at[idx_v.at[j]], rows_v.at[pl.ds(j * 128, 128)], sem
            )
            for j in range(8)
        ]
        for d in descs:
            d.wait()
        pltpu.sync_copy(rows_v, out_hbm.at[pl.ds(wid * 1024, 1024)])

    return gather_k(X, idx_sc)


def _tc_body(x1_ref, q_ref, idx_ref, rr_ref, s_ref, pre_ref, xd_ref):
    X1 = x1_ref[...]

    gs = []
    for q in range(_BQ):
        xq = X1[q]
        g = lax.dot_general(
            xq, xq, (((1,), (1,)), ((), ())),
            preferred_element_type=jnp.float32,
        )
        gs.append(g.reshape(1, _MP, _MP))
    G = jnp.concatenate(gs, axis=0)

    colmask = lax.broadcasted_iota(jnp.int32, (1, 1, _MP), 2) < _M
    GM = jnp.where(colmask, G, _NEG)

    Gf = GM.reshape(_BQ * _MP, _MP)
    GA = Gf[:, :, None]
    GB = Gf[:, None, :]
    jj = lax.broadcasted_iota(jnp.int32, (_MP, _MP), 0)
    kk = lax.broadcasted_iota(jnp.int32, (_MP, _MP), 1)
    prec = (kk < jj)[None]
    beats = (GB > GA) | ((GB == GA) & prec)
    rank1 = jnp.sum(beats.astype(jnp.int32), axis=2).reshape(_BQ, _MP, _MP)

    top1 = rank1 == 0
    sel = rank1 < _K
    W = jnp.where(top1, 1.0, jnp.where(sel, _BETA * G, 0.0))

    srows = []
    for q in range(_BQ):
        Wq = W[q]
        Xq = X1[q]
        dq = jnp.sum(Wq, axis=1, keepdims=True)
        nq = Wq[:, 0:1] * Xq[0:1, :]
        for j in range(1, _M):
            nq = nq + Wq[:, j:j + 1] * Xq[j:j + 1, :]
        xd = nq / dq
        xd_ref[q] = xd
        qrow = q_ref[q:q + 1, :]
        sq = jnp.sum(xd * qrow, axis=1, keepdims=True)
        srows.append(sq)
    S = jnp.concatenate(srows, axis=1)
    s_ref[0] = S

    imask = lax.broadcasted_iota(jnp.int32, (_MP, 1), 0) < _M
    SMv = jnp.where(imask, S, _NEG)
    SA = SMv[:, None, :]
    SB = SMv[None, :, :]
    i0 = lax.broadcasted_iota(jnp.int32, (_MP, _MP, 1), 0)
    i1 = lax.broadcasted_iota(jnp.int32, (_MP, _MP, 1), 1)
    prec2 = i1 < i0
    beats2 = (SB > SA) | ((SB == SA) & prec2)
    rank2 = jnp.sum(beats2.astype(jnp.int32), axis=1)

    O = rank2[:, None, :] == lax.broadcasted_iota(jnp.int32, (1, _MP, 1), 1)
    pre_ref[0] = jnp.sum(jnp.where(O, i0, 0), axis=0)
    idx3 = idx_ref[0][:, None, :]
    rr_ref[0] = jnp.sum(jnp.where(O, idx3, 0), axis=0)


def _tc_rerank(X1, Q, idxt3, interpret=False):
    nq = Q.shape[0]
    nb = nq // _BQ
    grid = (nb,)
    out_shapes = (
        jax.ShapeDtypeStruct((nb, _MP, _BQ), jnp.int32),
        jax.ShapeDtypeStruct((nb, _MP, _BQ), jnp.float32),
        jax.ShapeDtypeStruct((nb, _MP, _BQ), jnp.int32),
        jax.ShapeDtypeStruct((nq, _MP, 64), jnp.float32),
    )
    t3 = pl.BlockSpec((1, _MP, _BQ), lambda q: (q, 0, 0))
    return pl.pallas_call(
        _tc_body,
        grid=grid,
        in_specs=[
            pl.BlockSpec((_BQ, _MP, 64), lambda q: (q, 0, 0)),
            pl.BlockSpec((_BQ, 64), lambda q: (q, 0)),
            t3,
        ],
        out_specs=(t3, t3, t3,
                   pl.BlockSpec((_BQ, _MP, 64), lambda q: (q, 0, 0))),
        out_shape=out_shapes,
        interpret=interpret,
    )(X1, Q, idxt3)


def _untranspose(a3, nq):
    return a3.transpose(0, 2, 1).reshape(nq, _MP)


def kernel(X, Q, ranks):
    nq = Q.shape[0]
    idx30 = jnp.transpose(ranks[:_M, :])
    idxp = jnp.pad(idx30, ((0, 0), (0, _MP - _M)))
    idx_sc = idxp.reshape(_NW, 8, 128)
    idxt3 = idxp.reshape(nq // _BQ, _BQ, _MP).transpose(0, 2, 1)
    X1 = _sc_gather(X, idx_sc).reshape(-1, _MP, 64)
    rr3, s3, pre3, xd = _tc_rerank(X1, Q, idxt3)
    rr = _untranspose(rr3, nq)
    s = _untranspose(s3, nq)
    pre = _untranspose(pre3, nq)
    return (rr[:, :_M], s[:, :_M], pre[:, :_M], xd[:, :_M, :])
